# baseline (speedup 1.0000x reference)
import jax
import jax.numpy as jnp
from jax import lax
from jax.experimental import pallas as pl
from jax.experimental.pallas import tpu as pltpu
from jax.experimental.pallas import tpu_sc as plsc

B, H, W, V, F = 1, 1080, 1920, 100000, 200000
N = H * W
NC, NS = 2, 16
NW = NC * NS

F_PAD = 200704
CF = 32
S1_CHUNKS = F_PAD // (NS * CF)

GS = 128
NSUB = 4
CP = GS * NSUB
TOTAL_CHUNKS = N // CP
S2_CHUNKS = -(-TOTAL_CHUNKS // NW)

_params = pltpu.CompilerParams(
    use_tc_tiling_on_sc=False, needs_layout_passes=False)


def _mesh():
  return plsc.VectorSubcoreMesh(core_axis_name="c", subcore_axis_name="s")


def _worker_id():
  return lax.axis_index("s") * NC + lax.axis_index("c")


def _render(vpad8, faces3_flat, gidx2d, bary_planes):
  def body(vpad_hbm, fidx_hbm, gidx_hbm, bary_hbm, point_hbm, fattr2_hbm,
           idx_v, vrows_v, out_v, gidx_v, rows_v, bary_v, pt_v,
           sem_idx, sem_vrows, sem_gidx, sem_rows, sem_bary, sem_pt):
    cid = lax.axis_index("c")
    sid = lax.axis_index("s")
    wid = sid * NC + cid
    lane = lax.iota(jnp.int32, 16)
    fattr_hbm = fattr2_hbm.at[cid]

    def s1_start_idx(g, p):
      pltpu.async_copy(
          fidx_hbm.at[pl.ds((sid * S1_CHUNKS + g) * (3 * CF), 3 * CF)],
          idx_v.at[p], sem_idx.at[p])

    def s1_wait_idx(g, p):
      pltpu.make_async_copy(
          fidx_hbm.at[pl.ds((sid * S1_CHUNKS + g) * (3 * CF), 3 * CF)],
          idx_v.at[p], sem_idx.at[p]).wait()

    def s1_start_rows(p):
      pltpu.async_copy(vpad_hbm.at[idx_v.at[p]], vrows_v.at[p],
                       sem_vrows.at[p])

    def s1_wait_rows(p):
      pltpu.make_async_copy(vpad_hbm.at[idx_v.at[p]], vrows_v.at[p],
                            sem_vrows.at[p]).wait()

    s1_start_idx(0, 0)
    s1_wait_idx(0, 0)
    s1_start_rows(0)
    s1_start_idx(1, 1)

    @pl.loop(0, S1_CHUNKS)
    def _s1(g):
      p = lax.rem(g, 2)
      q = 1 - p

      @pl.when(g + 1 < S1_CHUNKS)
      def _():
        s1_wait_idx(g + 1, q)
        s1_start_rows(q)

      s1_wait_rows(p)
      for i in range(CF // 16):
        l = lane + (i * 16)
        l3 = l * 3
        for k in range(3):
          row = l3 + k
          for c in range(3):
            val = plsc.load_gather(vrows_v.at[p], [row, jnp.full((16,), c, jnp.int32)])
            plsc.store_scatter(out_v, [l, jnp.full((16,), 4 * k + c, jnp.int32)], val)
      pltpu.sync_copy(out_v, fattr_hbm.at[pl.ds((sid * S1_CHUNKS + g) * CF, CF)])

      @pl.when(g + 2 < S1_CHUNKS)
      def _():
        s1_start_idx(g + 2, p)

    plsc.subcore_barrier()

    def chunk_of(g):
      return g * NW + wid

    def start_gidx(t, p):
      pltpu.async_copy(gidx_hbm.at[pl.ds(t * NSUB, NSUB)], gidx_v.at[p],
                       sem_gidx.at[p])

    def wait_gidx(t, p):
      pltpu.make_async_copy(gidx_hbm.at[pl.ds(t * NSUB, NSUB)], gidx_v.at[p],
                            sem_gidx.at[p]).wait()

    def start_bary(t, p):
      for k in range(3):
        pltpu.async_copy(bary_hbm.at[k].at[pl.ds(t * CP, CP)],
                         bary_v.at[p].at[k], sem_bary.at[p])

    def wait_bary(t, p):
      for k in range(3):
        pltpu.make_async_copy(bary_hbm.at[k].at[pl.ds(t * CP, CP)],
                              bary_v.at[p].at[k], sem_bary.at[p]).wait()

    def start_rows(p):
      for j in range(NSUB):
        pltpu.async_copy(fattr_hbm.at[gidx_v.at[p].at[j]],
                         rows_v.at[p].at[j], sem_rows.at[p])

    def wait_rows(p):
      for j in range(NSUB):
        pltpu.make_async_copy(fattr_hbm.at[gidx_v.at[p].at[j]],
                              rows_v.at[p].at[j], sem_rows.at[p]).wait()

    def start_pt(t, p):
      for c in range(3):
        pltpu.async_copy(pt_v.at[p].at[c], point_hbm.at[c].at[pl.ds(t * CP, CP)],
                         sem_pt.at[p])

    def wait_pt(t, p):
      for c in range(3):
        pltpu.make_async_copy(pt_v.at[p].at[c],
                              point_hbm.at[c].at[pl.ds(t * CP, CP)],
                              sem_pt.at[p]).wait()

    start_gidx(chunk_of(0), 0)
    wait_gidx(chunk_of(0), 0)
    start_rows(0)
    start_bary(chunk_of(0), 0)
    start_gidx(chunk_of(1), 1)

    @pl.loop(0, S2_CHUNKS)
    def _chunk(g):
      p = lax.rem(g, 2)
      q = 1 - p
      t = chunk_of(g)

      @pl.when(chunk_of(g + 1) < TOTAL_CHUNKS)
      def _():
        wait_gidx(chunk_of(g + 1), q)
        start_rows(q)
        start_bary(chunk_of(g + 1), q)

      @pl.when(t < TOTAL_CHUNKS)
      def _():
        wait_rows(p)
        wait_bary(t, p)

        @pl.when(g >= 2)
        def _():
          wait_pt(chunk_of(g - 2), p)

        for jsub in range(NSUB):
          for i in range(GS // 16):
            o = jsub * GS + i * 16
            r = lane + (i * 16)
            b0 = bary_v[p, 0, pl.ds(o, 16)]
            b1 = bary_v[p, 1, pl.ds(o, 16)]
            b2 = bary_v[p, 2, pl.ds(o, 16)]
            for c in range(3):
              cc = jnp.full((16,), c, jnp.int32)
              v0 = plsc.load_gather(rows_v.at[p].at[jsub], [r, cc])
              v1 = plsc.load_gather(rows_v.at[p].at[jsub], [r, cc + 4])
              v2 = plsc.load_gather(rows_v.at[p].at[jsub], [r, cc + 8])
              pt_v[p, c, pl.ds(o, 16)] = b0 * v0 + b1 * v1 + b2 * v2
        start_pt(t, p)

      @pl.when(chunk_of(g + 2) < TOTAL_CHUNKS)
      def _():
        start_gidx(chunk_of(g + 2), p)

    for dg in (S2_CHUNKS - 2, S2_CHUNKS - 1):
      @pl.when(chunk_of(dg) < TOTAL_CHUNKS)
      def _(dg=dg):
        wait_pt(chunk_of(dg), dg % 2)

  return pl.kernel(
      body,
      out_type=(
          jax.ShapeDtypeStruct((3, N), jnp.float32),
          jax.ShapeDtypeStruct((2, F_PAD, 16), jnp.float32),
      ),
      mesh=_mesh(),
      compiler_params=_params,
      scratch_types=[
          pltpu.VMEM((2, 3 * CF), jnp.int32),
          pltpu.VMEM((2, 3 * CF, 8), jnp.float32),
          pltpu.VMEM((CF, 16), jnp.float32),
          pltpu.VMEM((2, NSUB, GS), jnp.int32),
          pltpu.VMEM((2, NSUB, GS, 16), jnp.float32),
          pltpu.VMEM((2, 3, CP), jnp.float32),
          pltpu.VMEM((2, 3, CP), jnp.float32),
          pltpu.SemaphoreType.DMA((2,)),
          pltpu.SemaphoreType.DMA((2,)),
          pltpu.SemaphoreType.DMA((2,)),
          pltpu.SemaphoreType.DMA((2,)),
          pltpu.SemaphoreType.DMA((2,)),
          pltpu.SemaphoreType.DMA((2,)),
      ],
  )(vpad8, faces3_flat, gidx2d, bary_planes)


def kernel(vertices, faces, pix_to_face, bary_coords):
  vpad8 = jnp.pad(vertices.reshape(V, 3), ((0, 1), (0, 5)))
  faces3 = jnp.pad(faces, ((0, F_PAD - F), (0, 0)),
                   constant_values=V)
  pix = pix_to_face.reshape(N)
  gidx2d = jnp.where(pix < 0, F, pix).reshape(N // GS, GS)
  bary_planes = jnp.moveaxis(bary_coords.reshape(N, 3), 1, 0)
  point_planes, _unused_fattr = _render(
      vpad8, faces3.reshape(-1), gidx2d, bary_planes)
  point = jnp.moveaxis(point_planes, 0, 1).reshape(B, H, W, 3)
  mask = pix_to_face != -1
  return point, mask


# speedup vs baseline: 1.7476x; 1.7476x over previous
import jax
import jax.numpy as jnp
from jax import lax
from jax.experimental import pallas as pl
from jax.experimental.pallas import tpu as pltpu
from jax.experimental.pallas import tpu_sc as plsc

B, H, W, V, F = 1, 1080, 1920, 100000, 200000
N = H * W
NC, NS = 2, 16
NW = NC * NS

F_PAD = 200704
CF = 32
S1_CHUNKS = F_PAD // (NW * CF)

GS = 128
NSUB = 4
CP = GS * NSUB
TOTAL_CHUNKS = N // CP
S2_CHUNKS = -(-TOTAL_CHUNKS // NW)
NT = N // GS

_params = pltpu.CompilerParams(
    use_tc_tiling_on_sc=False, needs_layout_passes=False)


def _mesh():
  return plsc.VectorSubcoreMesh(core_axis_name="c", subcore_axis_name="s")


def _worker_id():
  return lax.axis_index("s") * NC + lax.axis_index("c")


def _stage1(vpad8, faces3_flat):
  def body(vpad_hbm, fidx_hbm, fattr_hbm,
           idx_v, vrows_v, out_v, sem_idx, sem_rows):
    wid = _worker_id()
    lane = lax.iota(jnp.int32, 16)

    def start_idx(g, p):
      pltpu.async_copy(
          fidx_hbm.at[pl.ds((wid * S1_CHUNKS + g) * (3 * CF), 3 * CF)],
          idx_v.at[p], sem_idx.at[p])

    def wait_idx(g, p):
      pltpu.make_async_copy(
          fidx_hbm.at[pl.ds((wid * S1_CHUNKS + g) * (3 * CF), 3 * CF)],
          idx_v.at[p], sem_idx.at[p]).wait()

    def start_rows(p):
      pltpu.async_copy(vpad_hbm.at[idx_v.at[p]], vrows_v.at[p],
                       sem_rows.at[p])

    def wait_rows(p):
      pltpu.make_async_copy(vpad_hbm.at[idx_v.at[p]], vrows_v.at[p],
                            sem_rows.at[p]).wait()

    start_idx(0, 0)
    wait_idx(0, 0)
    start_rows(0)
    start_idx(1, 1)

    @pl.loop(0, S1_CHUNKS)
    def _chunk(g):
      p = lax.rem(g, 2)
      q = 1 - p

      @pl.when(g + 1 < S1_CHUNKS)
      def _():
        wait_idx(g + 1, q)
        start_rows(q)

      wait_rows(p)
      for i in range(CF // 16):
        l = lane + (i * 16)
        l3 = l * 3
        for k in range(3):
          row = l3 + k
          for c in range(3):
            val = plsc.load_gather(vrows_v.at[p], [row, jnp.full((16,), c, jnp.int32)])
            plsc.store_scatter(out_v, [l, jnp.full((16,), 4 * k + c, jnp.int32)], val)
      pltpu.sync_copy(out_v, fattr_hbm.at[pl.ds((wid * S1_CHUNKS + g) * CF, CF)])

      @pl.when(g + 2 < S1_CHUNKS)
      def _():
        start_idx(g + 2, p)

  return pl.kernel(
      body,
      out_type=jax.ShapeDtypeStruct((F_PAD, 16), jnp.float32),
      mesh=_mesh(),
      compiler_params=_params,
      scratch_types=[
          pltpu.VMEM((2, 3 * CF), jnp.int32),
          pltpu.VMEM((2, 3 * CF, 8), jnp.float32),
          pltpu.VMEM((CF, 16), jnp.float32),
          pltpu.SemaphoreType.DMA((2,)),
          pltpu.SemaphoreType.DMA((2,)),
      ],
  )(vpad8, faces3_flat)


def _stage2(fattr16, gidx2d, bary_planes):
  def body(fattr_hbm, gidx_hbm, bary_hbm, point_hbm,
           gidx_v, rows_v, bary_v, pt_v,
           sem_gidx, sem_rows, sem_bary, sem_pt):
    wid = _worker_id()
    lane = lax.iota(jnp.int32, 16)

    def chunk_of(g):
      return g * NW + wid

    def start_gidx(t, p):
      pltpu.async_copy(gidx_hbm.at[pl.ds(t * NSUB, NSUB)], gidx_v.at[p],
                       sem_gidx.at[p])

    def wait_gidx(t, p):
      pltpu.make_async_copy(gidx_hbm.at[pl.ds(t * NSUB, NSUB)], gidx_v.at[p],
                            sem_gidx.at[p]).wait()

    def start_bary(t, p):
      for k in range(3):
        pltpu.async_copy(bary_hbm.at[pl.ds(k * NT + t * NSUB, NSUB)],
                         bary_v.at[p].at[k], sem_bary.at[p])

    def wait_bary(t, p):
      for k in range(3):
        pltpu.make_async_copy(bary_hbm.at[pl.ds(k * NT + t * NSUB, NSUB)],
                              bary_v.at[p].at[k], sem_bary.at[p]).wait()

    def start_rows(p):
      for j in range(NSUB):
        pltpu.async_copy(fattr_hbm.at[gidx_v.at[p].at[j]],
                         rows_v.at[p].at[j], sem_rows.at[p])

    def wait_rows(p):
      for j in range(NSUB):
        pltpu.make_async_copy(fattr_hbm.at[gidx_v.at[p].at[j]],
                              rows_v.at[p].at[j], sem_rows.at[p]).wait()

    def start_pt(t, p):
      for c in range(3):
        pltpu.async_copy(pt_v.at[p].at[c],
                         point_hbm.at[pl.ds(c * NT + t * NSUB, NSUB)],
                         sem_pt.at[p])

    def wait_pt(t, p):
      for c in range(3):
        pltpu.make_async_copy(pt_v.at[p].at[c],
                              point_hbm.at[pl.ds(c * NT + t * NSUB, NSUB)],
                              sem_pt.at[p]).wait()

    start_gidx(chunk_of(0), 0)
    wait_gidx(chunk_of(0), 0)
    start_rows(0)
    start_bary(chunk_of(0), 0)
    start_gidx(chunk_of(1), 1)

    @pl.loop(0, S2_CHUNKS)
    def _chunk(g):
      p = lax.rem(g, 2)
      q = 1 - p
      t = chunk_of(g)

      @pl.when(chunk_of(g + 1) < TOTAL_CHUNKS)
      def _():
        wait_gidx(chunk_of(g + 1), q)
        start_rows(q)
        start_bary(chunk_of(g + 1), q)

      @pl.when(t < TOTAL_CHUNKS)
      def _():
        wait_rows(p)
        wait_bary(t, p)

        @pl.when(g >= 2)
        def _():
          wait_pt(chunk_of(g - 2), p)

        for jsub in range(NSUB):
          for i in range(GS // 16):
            o = jsub * GS + i * 16
            r = lane + (i * 16)
            oi = i * 16
            b0 = bary_v[p, 0, jsub, pl.ds(oi, 16)]
            b1 = bary_v[p, 1, jsub, pl.ds(oi, 16)]
            b2 = bary_v[p, 2, jsub, pl.ds(oi, 16)]
            for c in range(3):
              cc = jnp.full((16,), c, jnp.int32)
              v0 = plsc.load_gather(rows_v.at[p].at[jsub], [r, cc])
              v1 = plsc.load_gather(rows_v.at[p].at[jsub], [r, cc + 4])
              v2 = plsc.load_gather(rows_v.at[p].at[jsub], [r, cc + 8])
              pt_v[p, c, jsub, pl.ds(oi, 16)] = b0 * v0 + b1 * v1 + b2 * v2
        start_pt(t, p)

      @pl.when(chunk_of(g + 2) < TOTAL_CHUNKS)
      def _():
        start_gidx(chunk_of(g + 2), p)

    for dg in (S2_CHUNKS - 2, S2_CHUNKS - 1):
      @pl.when(chunk_of(dg) < TOTAL_CHUNKS)
      def _(dg=dg):
        wait_pt(chunk_of(dg), dg % 2)

  return pl.kernel(
      body,
      out_type=jax.ShapeDtypeStruct((3 * NT, GS), jnp.float32),
      mesh=_mesh(),
      compiler_params=_params,
      scratch_types=[
          pltpu.VMEM((2, NSUB, GS), jnp.int32),
          pltpu.VMEM((2, NSUB, GS, 16), jnp.float32),
          pltpu.VMEM((2, 3, NSUB, GS), jnp.float32),
          pltpu.VMEM((2, 3, NSUB, GS), jnp.float32),
          pltpu.SemaphoreType.DMA((2,)),
          pltpu.SemaphoreType.DMA((2,)),
          pltpu.SemaphoreType.DMA((2,)),
          pltpu.SemaphoreType.DMA((2,)),
      ],
  )(fattr16, gidx2d, bary_planes)


def kernel(vertices, faces, pix_to_face, bary_coords):
  vpad8 = jnp.pad(vertices.reshape(V, 3), ((0, 1), (0, 5)))
  faces3 = jnp.pad(faces, ((0, F_PAD - F), (0, 0)),
                   constant_values=V)
  fattr16 = _stage1(vpad8, faces3.reshape(-1))
  pix = pix_to_face.reshape(N)
  gidx2d = jnp.where(pix < 0, F, pix).reshape(N // GS, GS)
  bary_planes = jnp.moveaxis(bary_coords.reshape(N, 3), 1, 0)
  point_planes = _stage2(
      fattr16, gidx2d, bary_planes.reshape(3 * NT, GS))
  point = jnp.moveaxis(point_planes.reshape(3, N), 0, 1).reshape(B, H, W, 3)
  mask = pix_to_face != -1
  return point, mask


# speedup vs baseline: 2.7216x; 1.5574x over previous
import jax
import jax.numpy as jnp
from jax import lax
from jax.experimental import pallas as pl
from jax.experimental.pallas import tpu as pltpu
from jax.experimental.pallas import tpu_sc as plsc

B, H, W, V, F = 1, 1080, 1920, 100000, 200000
N = H * W
NC, NS = 2, 16
NW = NC * NS

F_PAD = 200704
CF = 32
S1_CHUNKS = F_PAD // (NW * CF)

GS = 128
NSUB = 4
CP = GS * NSUB
TOTAL_CHUNKS = N // CP
S2_CHUNKS = -(-TOTAL_CHUNKS // NW)
NT = N // GS

_params = pltpu.CompilerParams(
    use_tc_tiling_on_sc=False, needs_layout_passes=False)


def _mesh():
  return plsc.VectorSubcoreMesh(core_axis_name="c", subcore_axis_name="s")


def _worker_id():
  return lax.axis_index("s") * NC + lax.axis_index("c")


def _stage1(vpad8, faces3_flat):
  def body(vpad_hbm, fidx_hbm, fattr_hbm,
           idx_v, vrows_v, out_v, sem_idx, sem_rows):
    wid = _worker_id()
    lane = lax.iota(jnp.int32, 16)

    def start_idx(g, p):
      pltpu.async_copy(
          fidx_hbm.at[pl.ds((wid * S1_CHUNKS + g) * (3 * CF), 3 * CF)],
          idx_v.at[p], sem_idx.at[p])

    def wait_idx(g, p):
      pltpu.make_async_copy(
          fidx_hbm.at[pl.ds((wid * S1_CHUNKS + g) * (3 * CF), 3 * CF)],
          idx_v.at[p], sem_idx.at[p]).wait()

    def start_rows(p):
      pltpu.async_copy(vpad_hbm.at[idx_v.at[p]], vrows_v.at[p],
                       sem_rows.at[p])

    def wait_rows(p):
      pltpu.make_async_copy(vpad_hbm.at[idx_v.at[p]], vrows_v.at[p],
                            sem_rows.at[p]).wait()

    start_idx(0, 0)
    wait_idx(0, 0)
    start_rows(0)
    start_idx(1, 1)

    @pl.loop(0, S1_CHUNKS)
    def _chunk(g):
      p = lax.rem(g, 2)
      q = 1 - p

      @pl.when(g + 1 < S1_CHUNKS)
      def _():
        wait_idx(g + 1, q)
        start_rows(q)

      wait_rows(p)
      for i in range(CF // 16):
        l = lane + (i * 16)
        l3 = l * 3
        for k in range(3):
          row = l3 + k
          for c in range(3):
            val = plsc.load_gather(vrows_v.at[p], [row, jnp.full((16,), c, jnp.int32)])
            plsc.store_scatter(out_v, [l, jnp.full((16,), 4 * k + c, jnp.int32)], val)
      pltpu.sync_copy(out_v, fattr_hbm.at[pl.ds((wid * S1_CHUNKS + g) * CF, CF)])

      @pl.when(g + 2 < S1_CHUNKS)
      def _():
        start_idx(g + 2, p)

  return pl.kernel(
      body,
      out_type=jax.ShapeDtypeStruct((F_PAD, 16), jnp.float32),
      mesh=_mesh(),
      compiler_params=_params,
      scratch_types=[
          pltpu.VMEM((2, 3 * CF), jnp.int32),
          pltpu.VMEM((2, 3 * CF, 8), jnp.float32),
          pltpu.VMEM((CF, 16), jnp.float32),
          pltpu.SemaphoreType.DMA((2,)),
          pltpu.SemaphoreType.DMA((2,)),
      ],
  )(vpad8, faces3_flat)


def _stage2(fattr16, gidx2d, bary_planes):
  def body(fattr_hbm, gidx_hbm, bary_hbm, point_hbm,
           gidx_v, rows_v, bary_v, pt_v,
           sem_gidx, sem_rows, sem_bary, sem_pt):
    wid = _worker_id()
    lane = lax.iota(jnp.int32, 16)

    def chunk_of(g):
      return g * NW + wid

    def start_gidx(t, p):
      pltpu.async_copy(gidx_hbm.at[pl.ds(t * NSUB, NSUB)], gidx_v.at[p],
                       sem_gidx.at[p])

    def wait_gidx(t, p):
      pltpu.make_async_copy(gidx_hbm.at[pl.ds(t * NSUB, NSUB)], gidx_v.at[p],
                            sem_gidx.at[p]).wait()

    def start_bary(t, p):
      for k in range(3):
        pltpu.async_copy(bary_hbm.at[pl.ds(k * NT + t * NSUB, NSUB)],
                         bary_v.at[p].at[k], sem_bary.at[p])

    def wait_bary(t, p):
      for k in range(3):
        pltpu.make_async_copy(bary_hbm.at[pl.ds(k * NT + t * NSUB, NSUB)],
                              bary_v.at[p].at[k], sem_bary.at[p]).wait()

    def start_rows(p):
      for j in range(NSUB):
        pltpu.async_copy(fattr_hbm.at[gidx_v.at[p].at[j]],
                         rows_v.at[p].at[j], sem_rows.at[p])

    def wait_rows(p):
      for j in range(NSUB):
        pltpu.make_async_copy(fattr_hbm.at[gidx_v.at[p].at[j]],
                              rows_v.at[p].at[j], sem_rows.at[p]).wait()

    def start_pt(t, p):
      for c in range(3):
        pltpu.async_copy(pt_v.at[p].at[c],
                         point_hbm.at[pl.ds(c * NT + t * NSUB, NSUB)],
                         sem_pt.at[p])

    def wait_pt(t, p):
      for c in range(3):
        pltpu.make_async_copy(pt_v.at[p].at[c],
                              point_hbm.at[pl.ds(c * NT + t * NSUB, NSUB)],
                              sem_pt.at[p]).wait()

    start_gidx(chunk_of(0), 0)
    wait_gidx(chunk_of(0), 0)
    start_rows(0)
    start_bary(chunk_of(0), 0)
    start_gidx(chunk_of(1), 1)

    @pl.loop(0, S2_CHUNKS)
    def _chunk(g):
      p = lax.rem(g, 2)
      q = 1 - p
      t = chunk_of(g)

      @pl.when(chunk_of(g + 1) < TOTAL_CHUNKS)
      def _():
        wait_gidx(chunk_of(g + 1), q)
        start_rows(q)
        start_bary(chunk_of(g + 1), q)

      @pl.when(t < TOTAL_CHUNKS)
      def _():
        wait_rows(p)
        wait_bary(t, p)

        @pl.when(g >= 2)
        def _():
          wait_pt(chunk_of(g - 2), p)

        for jsub in range(NSUB):
          for i in range(GS // 16):
            o = jsub * GS + i * 16
            r = lane + (i * 16)
            oi = i * 16
            b0 = bary_v[p, 0, jsub, pl.ds(oi, 16)]
            b1 = bary_v[p, 1, jsub, pl.ds(oi, 16)]
            b2 = bary_v[p, 2, jsub, pl.ds(oi, 16)]
            for c in range(3):
              cc = jnp.full((16,), c, jnp.int32)
              v0 = plsc.load_gather(rows_v.at[p].at[jsub], [r, cc])
              v1 = plsc.load_gather(rows_v.at[p].at[jsub], [r, cc + 4])
              v2 = plsc.load_gather(rows_v.at[p].at[jsub], [r, cc + 8])
              pt_v[p, c, jsub, pl.ds(oi, 16)] = b0 * v0 + b1 * v1 + b2 * v2
        start_pt(t, p)

      @pl.when(chunk_of(g + 2) < TOTAL_CHUNKS)
      def _():
        start_gidx(chunk_of(g + 2), p)

    for dg in (S2_CHUNKS - 2, S2_CHUNKS - 1):
      @pl.when(chunk_of(dg) < TOTAL_CHUNKS)
      def _(dg=dg):
        wait_pt(chunk_of(dg), dg % 2)

  return pl.kernel(
      body,
      out_type=jax.ShapeDtypeStruct((3 * NT, GS), jnp.float32),
      mesh=_mesh(),
      compiler_params=_params,
      scratch_types=[
          pltpu.VMEM((2, NSUB, GS), jnp.int32),
          pltpu.VMEM((2, NSUB, GS, 16), jnp.float32),
          pltpu.VMEM((2, 3, NSUB, GS), jnp.float32),
          pltpu.VMEM((2, 3, NSUB, GS), jnp.float32),
          pltpu.SemaphoreType.DMA((2,)),
          pltpu.SemaphoreType.DMA((2,)),
          pltpu.SemaphoreType.DMA((2,)),
          pltpu.SemaphoreType.DMA((2,)),
      ],
  )(fattr16, gidx2d, bary_planes)


_HT, _WT = H // 8, W // GS


def _tile_order(x):
  return x.reshape(_HT, 8, _WT, GS).transpose(0, 2, 1, 3).reshape(NT, GS)


def _untile(y):
  return y.reshape(_HT, _WT, 8, GS).transpose(0, 2, 1, 3).reshape(H, W)


def kernel(vertices, faces, pix_to_face, bary_coords):
  vpad8 = jnp.pad(vertices.reshape(V, 3), ((0, 1), (0, 5)))
  faces3 = jnp.pad(faces, ((0, F_PAD - F), (0, 0)),
                   constant_values=V)
  fattr16 = _stage1(vpad8, faces3.reshape(-1))
  gidx2d = _tile_order(jnp.where(pix_to_face < 0, F,
                                 pix_to_face).reshape(H, W))
  bary_r = bary_coords.reshape(H, W, 3)
  bary_planes = jnp.concatenate(
      [_tile_order(bary_r[:, :, k]) for k in range(3)], axis=0)
  point_planes = _stage2(fattr16, gidx2d, bary_planes)
  point = jnp.stack(
      [_untile(point_planes[c * NT:(c + 1) * NT]) for c in range(3)],
      axis=-1).reshape(B, H, W, 3)
  mask = pix_to_face != -1
  return point, mask


# speedup vs baseline: 2.9643x; 1.0892x over previous
import jax
import jax.numpy as jnp
from jax import lax
from jax.experimental import pallas as pl
from jax.experimental.pallas import tpu as pltpu
from jax.experimental.pallas import tpu_sc as plsc

B, H, W, V, F = 1, 1080, 1920, 100000, 200000
N = H * W
NC, NS = 2, 16
NW = NC * NS

F_PAD = 200704
CF = 32
S1_CHUNKS = F_PAD // (NW * CF)

GS = 128
NSUB = 4
CP = GS * NSUB
TOTAL_CHUNKS = N // CP
S2_CHUNKS = -(-TOTAL_CHUNKS // NW)
NT = N // GS

_params = pltpu.CompilerParams(
    use_tc_tiling_on_sc=False, needs_layout_passes=False)


def _mesh():
  return plsc.VectorSubcoreMesh(core_axis_name="c", subcore_axis_name="s")


def _worker_id():
  return lax.axis_index("s") * NC + lax.axis_index("c")


def _stage1(vpad8, faces3_flat):
  def body(vpad_hbm, fidx_hbm, fattr_hbm,
           idx_v, vrows_v, out_v, sem_idx, sem_rows, sem_out):
    wid = _worker_id()
    lane = lax.iota(jnp.int32, 16)

    def start_idx(g):
      pltpu.async_copy(
          fidx_hbm.at[pl.ds((wid * S1_CHUNKS + g) * (3 * CF), 3 * CF)],
          idx_v.at[lax.rem(g, 4)], sem_idx.at[lax.rem(g, 4)])

    def wait_idx(g):
      pltpu.make_async_copy(
          fidx_hbm.at[pl.ds((wid * S1_CHUNKS + g) * (3 * CF), 3 * CF)],
          idx_v.at[lax.rem(g, 4)], sem_idx.at[lax.rem(g, 4)]).wait()

    def start_rows(g):
      pltpu.async_copy(vpad_hbm.at[idx_v.at[lax.rem(g, 4)]],
                       vrows_v.at[lax.rem(g, 4)], sem_rows.at[lax.rem(g, 4)])

    def wait_rows(g):
      pltpu.make_async_copy(vpad_hbm.at[idx_v.at[lax.rem(g, 4)]],
                            vrows_v.at[lax.rem(g, 4)],
                            sem_rows.at[lax.rem(g, 4)]).wait()

    def start_out(g):
      pltpu.async_copy(out_v.at[lax.rem(g, 2)],
                       fattr_hbm.at[pl.ds((wid * S1_CHUNKS + g) * CF, CF)],
                       sem_out.at[lax.rem(g, 2)])

    def wait_out(g):
      pltpu.make_async_copy(out_v.at[lax.rem(g, 2)],
                            fattr_hbm.at[pl.ds((wid * S1_CHUNKS + g) * CF, CF)],
                            sem_out.at[lax.rem(g, 2)]).wait()

    start_idx(0)
    start_idx(1)
    wait_idx(0)
    start_rows(0)
    wait_idx(1)
    start_rows(1)
    start_idx(2)
    start_idx(3)

    @pl.loop(0, S1_CHUNKS)
    def _chunk(g):
      @pl.when(g + 2 < S1_CHUNKS)
      def _():
        wait_idx(g + 2)
        start_rows(g + 2)

      wait_rows(g)

      @pl.when(g >= 2)
      def _():
        wait_out(g - 2)

      p4 = lax.rem(g, 4)
      p2 = lax.rem(g, 2)
      for i in range(CF // 16):
        l = lane + (i * 16)
        l3 = l * 3
        for k in range(3):
          row = l3 + k
          for c in range(3):
            val = plsc.load_gather(vrows_v.at[p4], [row, jnp.full((16,), c, jnp.int32)])
            plsc.store_scatter(out_v.at[p2], [l, jnp.full((16,), 4 * k + c, jnp.int32)], val)
      start_out(g)

      @pl.when(g + 4 < S1_CHUNKS)
      def _():
        start_idx(g + 4)

    wait_out(S1_CHUNKS - 2)
    wait_out(S1_CHUNKS - 1)

  return pl.kernel(
      body,
      out_type=jax.ShapeDtypeStruct((F_PAD, 16), jnp.float32),
      mesh=_mesh(),
      compiler_params=_params,
      scratch_types=[
          pltpu.VMEM((4, 3 * CF), jnp.int32),
          pltpu.VMEM((4, 3 * CF, 8), jnp.float32),
          pltpu.VMEM((2, CF, 16), jnp.float32),
          pltpu.SemaphoreType.DMA((4,)),
          pltpu.SemaphoreType.DMA((4,)),
          pltpu.SemaphoreType.DMA((2,)),
      ],
  )(vpad8, faces3_flat)


def _stage2(fattr16, gidx2d, bary_planes):
  def body(fattr_hbm, gidx_hbm, bary_hbm, point_hbm,
           gidx_v, rows_v, bary_v, pt_v,
           sem_gidx, sem_rows, sem_bary, sem_pt):
    wid = _worker_id()
    lane = lax.iota(jnp.int32, 16)

    def chunk_of(g):
      return g * NW + wid

    def start_gidx(t, p):
      pltpu.async_copy(gidx_hbm.at[pl.ds(t * NSUB, NSUB)], gidx_v.at[p],
                       sem_gidx.at[p])

    def wait_gidx(t, p):
      pltpu.make_async_copy(gidx_hbm.at[pl.ds(t * NSUB, NSUB)], gidx_v.at[p],
                            sem_gidx.at[p]).wait()

    def start_bary(t, p):
      for k in range(3):
        pltpu.async_copy(bary_hbm.at[pl.ds(k * NT + t * NSUB, NSUB)],
                         bary_v.at[p].at[k], sem_bary.at[p])

    def wait_bary(t, p):
      for k in range(3):
        pltpu.make_async_copy(bary_hbm.at[pl.ds(k * NT + t * NSUB, NSUB)],
                              bary_v.at[p].at[k], sem_bary.at[p]).wait()

    def start_rows(p):
      for j in range(NSUB):
        pltpu.async_copy(fattr_hbm.at[gidx_v.at[p].at[j]],
                         rows_v.at[p].at[j], sem_rows.at[p])

    def wait_rows(p):
      for j in range(NSUB):
        pltpu.make_async_copy(fattr_hbm.at[gidx_v.at[p].at[j]],
                              rows_v.at[p].at[j], sem_rows.at[p]).wait()

    def start_pt(t, p):
      for c in range(3):
        pltpu.async_copy(pt_v.at[p].at[c],
                         point_hbm.at[pl.ds(c * NT + t * NSUB, NSUB)],
                         sem_pt.at[p])

    def wait_pt(t, p):
      for c in range(3):
        pltpu.make_async_copy(pt_v.at[p].at[c],
                              point_hbm.at[pl.ds(c * NT + t * NSUB, NSUB)],
                              sem_pt.at[p]).wait()

    start_gidx(chunk_of(0), 0)
    wait_gidx(chunk_of(0), 0)
    start_rows(0)
    start_bary(chunk_of(0), 0)
    start_gidx(chunk_of(1), 1)

    @pl.loop(0, S2_CHUNKS)
    def _chunk(g):
      p = lax.rem(g, 2)
      q = 1 - p
      t = chunk_of(g)

      @pl.when(chunk_of(g + 1) < TOTAL_CHUNKS)
      def _():
        wait_gidx(chunk_of(g + 1), q)
        start_rows(q)
        start_bary(chunk_of(g + 1), q)

      @pl.when(t < TOTAL_CHUNKS)
      def _():
        wait_rows(p)
        wait_bary(t, p)

        @pl.when(g >= 2)
        def _():
          wait_pt(chunk_of(g - 2), p)

        for jsub in range(NSUB):
          for i in range(GS // 16):
            o = jsub * GS + i * 16
            r = lane + (i * 16)
            oi = i * 16
            b0 = bary_v[p, 0, jsub, pl.ds(oi, 16)]
            b1 = bary_v[p, 1, jsub, pl.ds(oi, 16)]
            b2 = bary_v[p, 2, jsub, pl.ds(oi, 16)]
            for c in range(3):
              cc = jnp.full((16,), c, jnp.int32)
              v0 = plsc.load_gather(rows_v.at[p].at[jsub], [r, cc])
              v1 = plsc.load_gather(rows_v.at[p].at[jsub], [r, cc + 4])
              v2 = plsc.load_gather(rows_v.at[p].at[jsub], [r, cc + 8])
              pt_v[p, c, jsub, pl.ds(oi, 16)] = b0 * v0 + b1 * v1 + b2 * v2
        start_pt(t, p)

      @pl.when(chunk_of(g + 2) < TOTAL_CHUNKS)
      def _():
        start_gidx(chunk_of(g + 2), p)

    for dg in (S2_CHUNKS - 2, S2_CHUNKS - 1):
      @pl.when(chunk_of(dg) < TOTAL_CHUNKS)
      def _(dg=dg):
        wait_pt(chunk_of(dg), dg % 2)

  return pl.kernel(
      body,
      out_type=jax.ShapeDtypeStruct((3 * NT, GS), jnp.float32),
      mesh=_mesh(),
      compiler_params=_params,
      scratch_types=[
          pltpu.VMEM((2, NSUB, GS), jnp.int32),
          pltpu.VMEM((2, NSUB, GS, 16), jnp.float32),
          pltpu.VMEM((2, 3, NSUB, GS), jnp.float32),
          pltpu.VMEM((2, 3, NSUB, GS), jnp.float32),
          pltpu.SemaphoreType.DMA((2,)),
          pltpu.SemaphoreType.DMA((2,)),
          pltpu.SemaphoreType.DMA((2,)),
          pltpu.SemaphoreType.DMA((2,)),
      ],
  )(fattr16, gidx2d, bary_planes)


_HT, _WT = H // 8, W // GS


def _tile_order(x):
  return x.reshape(_HT, 8, _WT, GS).transpose(0, 2, 1, 3).reshape(NT, GS)


def _untile(y):
  return y.reshape(_HT, _WT, 8, GS).transpose(0, 2, 1, 3).reshape(H, W)


def kernel(vertices, faces, pix_to_face, bary_coords):
  vpad8 = jnp.pad(vertices.reshape(V, 3), ((0, 1), (0, 5)))
  faces3 = jnp.pad(faces, ((0, F_PAD - F), (0, 0)),
                   constant_values=V)
  fattr16 = _stage1(vpad8, faces3.reshape(-1))
  gidx2d = _tile_order(jnp.where(pix_to_face < 0, F,
                                 pix_to_face).reshape(H, W))
  bary_r = bary_coords.reshape(H, W, 3)
  bary_planes = jnp.concatenate(
      [_tile_order(bary_r[:, :, k]) for k in range(3)], axis=0)
  point_planes = _stage2(fattr16, gidx2d, bary_planes)
  point = jnp.stack(
      [_untile(point_planes[c * NT:(c + 1) * NT]) for c in range(3)],
      axis=-1).reshape(B, H, W, 3)
  mask = pix_to_face != -1
  return point, mask
